# initial kernel scaffold (unmeasured)
import jax
import jax.numpy as jnp
from jax import lax
from jax.experimental import pallas as pl
from jax.experimental.pallas import tpu as pltpu


def kernel(
    x,
):
    def body(*refs):
        pass

    out_shape = jax.ShapeDtypeStruct(..., jnp.float32)
    return pl.pallas_call(body, out_shape=out_shape)(...)


# baseline (device time: 112705 ns/iter reference)
import functools

import jax
import jax.numpy as jnp
from jax import lax
from jax.experimental import pallas as pl
from jax.experimental.pallas import tpu as pltpu

N_DEV = 8


def kernel(x):
    m, n = x.shape
    mc = m // N_DEV

    def body(x_ref, out_ref, rs_recv, rs_send,
             rs_send_sems, rs_recv_sems, ag_send_sems, ag_recv_sems):
        my = lax.axis_index("i")
        left = (my + N_DEV - 1) % N_DEV
        right = (my + 1) % N_DEV

        barrier_sem = pltpu.get_barrier_semaphore()
        for nbr in (left, right):
            pl.semaphore_signal(
                barrier_sem, inc=1,
                device_id=(nbr,), device_id_type=pl.DeviceIdType.MESH,
            )
        pl.semaphore_wait(barrier_sem, 2)

        for s in range(N_DEV - 1):
            c_send = (my + N_DEV - s) % N_DEV
            if s == 0:
                src = x_ref.at[pl.ds(c_send * mc, mc), :]
            else:
                rs_send[s, :, :] = (
                    rs_recv[s - 1, :, :]
                    + x_ref[pl.ds(c_send * mc, mc), :]
                )
                src = rs_send.at[s]
            rdma = pltpu.make_async_remote_copy(
                src_ref=src,
                dst_ref=rs_recv.at[s],
                send_sem=rs_send_sems.at[s],
                recv_sem=rs_recv_sems.at[s],
                device_id=(right,),
                device_id_type=pl.DeviceIdType.MESH,
            )
            rdma.start()
            rdma.wait()

        c_own = (my + 1) % N_DEV
        out_ref[pl.ds(c_own * mc, mc), :] = (
            rs_recv[N_DEV - 2, :, :] + x_ref[pl.ds(c_own * mc, mc), :]
        )

        for t in range(N_DEV - 1):
            c_send = (my + N_DEV + 1 - t) % N_DEV
            c_recv = (my + N_DEV - t) % N_DEV
            rdma = pltpu.make_async_remote_copy(
                src_ref=out_ref.at[pl.ds(c_send * mc, mc), :],
                dst_ref=out_ref.at[pl.ds(c_send * mc, mc), :],
                send_sem=ag_send_sems.at[t],
                recv_sem=ag_recv_sems.at[t],
                device_id=(right,),
                device_id_type=pl.DeviceIdType.MESH,
            )
            rdma.start()
            rdma.wait()

        @functools.partial(
            pl.run_scoped, second_barrier=pltpu.SemaphoreType.REGULAR
        )
        def _(second_barrier):
            for nbr in (left, right):
                pl.semaphore_signal(
                    second_barrier, inc=1,
                    device_id=(nbr,), device_id_type=pl.DeviceIdType.MESH,
                )
            pl.semaphore_wait(second_barrier, 2)

    return pl.pallas_call(
        body,
        out_shape=jax.ShapeDtypeStruct((m, n), x.dtype),
        in_specs=[pl.BlockSpec(memory_space=pltpu.VMEM)],
        out_specs=pl.BlockSpec(memory_space=pltpu.VMEM),
        scratch_shapes=[
            pltpu.VMEM((N_DEV - 1, mc, n), x.dtype),
            pltpu.VMEM((N_DEV - 1, mc, n), x.dtype),
            pltpu.SemaphoreType.DMA((N_DEV - 1,)),
            pltpu.SemaphoreType.DMA((N_DEV - 1,)),
            pltpu.SemaphoreType.DMA((N_DEV - 1,)),
            pltpu.SemaphoreType.DMA((N_DEV - 1,)),
        ],
        compiler_params=pltpu.CompilerParams(collective_id=0),
    )(x)


# device time: 49141 ns/iter; 2.2935x vs baseline; 2.2935x over previous
import jax
import jax.numpy as jnp
from jax import lax
from jax.experimental import pallas as pl
from jax.experimental.pallas import tpu as pltpu

N_DEV = 8
MASKS = (1, 3, 4)


def _span(gens):
    s = {0}
    for g in gens:
        s |= {a ^ g for a in s}
    return sorted(s)


_BFLY = (
    (0, 96, (1, 3, 4)),
    (768, 96, (3, 4, 1)),
    (1536, 64, (4, 1, 3)),
)

_H = [
    {1: _span(masks[1:]), 2: _span(masks[2:]), 3: [0]}
    for (_, _, masks) in _BFLY
]
_O = [
    {1: [0], 2: _span(masks[2:]), 3: _span(masks[1:])}
    for (_, _, masks) in _BFLY
]

_RS_SLOT = {1: 0, 2: 4, 3: 6}
_AG_SLOT = {1: 7, 2: 8, 3: 10}


def kernel(x):
    m, n = x.shape

    def body(x_ref, out_ref, r1, r2, r3, send_sems, recv_sems):
        my = lax.axis_index("i")
        recv_bufs = {1: r1, 2: r2, 3: r3}

        barrier_sem = pltpu.get_barrier_semaphore()
        for mask in MASKS:
            pl.semaphore_signal(
                barrier_sem, inc=1,
                device_id=(my ^ mask,), device_id_type=pl.DeviceIdType.MESH,
            )
        pl.semaphore_wait(barrier_sem, len(MASKS))

        out_ref[:, :] = x_ref[:, :]

        def desc(b, base, r, c, dst_buf, slot, target):
            rows = pl.ds(base + c * r, r)
            return pltpu.make_async_remote_copy(
                src_ref=out_ref.at[rows, :],
                dst_ref=dst_buf.at[rows, :],
                send_sem=send_sems.at[b, slot],
                recv_sem=recv_sems.at[b, slot],
                device_id=(target,),
                device_id_type=pl.DeviceIdType.MESH,
            )

        for k in (1, 2, 3):
            started = []
            for b, (base, r, masks) in enumerate(_BFLY):
                mask = masks[k - 1]
                for i, h in enumerate(_H[b][k]):
                    d = desc(b, base, r, my ^ mask ^ h, recv_bufs[k],
                             _RS_SLOT[k] + i, my ^ mask)
                    d.start()
                    started.append(d)
            for d in started:
                d.wait_recv()
            for b, (base, r, masks) in enumerate(_BFLY):
                for h in _H[b][k]:
                    c = my ^ h
                    rows = pl.ds(base + c * r, r)
                    out_ref[rows, :] = (
                        out_ref[rows, :] + recv_bufs[k][rows, :]
                    )
            for d in started:
                d.wait_send()

        for j in (1, 2, 3):
            started = []
            for b, (base, r, masks) in enumerate(_BFLY):
                mask = masks[3 - j]
                for i, o in enumerate(_O[b][j]):
                    d = desc(b, base, r, my ^ o, out_ref,
                             _AG_SLOT[j] + i, my ^ mask)
                    d.start()
                    started.append(d)
            for d in started:
                d.wait_recv()
            for d in started:
                d.wait_send()

    return pl.pallas_call(
        body,
        out_shape=jax.ShapeDtypeStruct((m, n), x.dtype),
        in_specs=[pl.BlockSpec(memory_space=pltpu.VMEM)],
        out_specs=pl.BlockSpec(memory_space=pltpu.VMEM),
        scratch_shapes=[
            pltpu.VMEM((m, n), x.dtype),
            pltpu.VMEM((m, n), x.dtype),
            pltpu.VMEM((m, n), x.dtype),
            pltpu.SemaphoreType.DMA((3, 14)),
            pltpu.SemaphoreType.DMA((3, 14)),
        ],
        compiler_params=pltpu.CompilerParams(collective_id=0),
    )(x)


# device time: 44420 ns/iter; 2.5373x vs baseline; 1.1063x over previous
import jax
import jax.numpy as jnp
from jax import lax
from jax.experimental import pallas as pl
from jax.experimental.pallas import tpu as pltpu

N_DEV = 8
MASKS = (1, 3, 4)


def _span(gens):
    s = {0}
    for g in gens:
        s |= {a ^ g for a in s}
    return sorted(s)


_BFLY = (
    (0, 88, (1, 3, 4)),
    (704, 80, (3, 4, 1)),
    (1344, 88, (4, 1, 3)),
)

_H = [
    {1: _span(masks[1:]), 2: _span(masks[2:])}
    for (_, _, masks) in _BFLY
]
_O = [
    {2: _span(masks[2:]), 3: _span(masks[1:])}
    for (_, _, masks) in _BFLY
]

_RS_SLOT = {1: 0, 2: 4}
_MERGE_SLOT = 6
_AG_SLOT = {2: 8, 3: 10}


def kernel(x):
    m, n = x.shape

    def body(x_ref, out_ref, r1, r2, r3, send_sems, recv_sems):
        my = lax.axis_index("i")
        recv_bufs = {1: r1, 2: r2, 3: r3}

        barrier_sem = pltpu.get_barrier_semaphore()
        for mask in MASKS:
            pl.semaphore_signal(
                barrier_sem, inc=1,
                device_id=(my ^ mask,), device_id_type=pl.DeviceIdType.MESH,
            )
        pl.semaphore_wait(barrier_sem, len(MASKS))

        def desc(src_buf, dst_buf, base, r, c, b, slot, target):
            rows = pl.ds(base + c * r, r)
            return pltpu.make_async_remote_copy(
                src_ref=src_buf.at[rows, :],
                dst_ref=dst_buf.at[rows, :],
                send_sem=send_sems.at[b, slot],
                recv_sem=recv_sems.at[b, slot],
                device_id=(target,),
                device_id_type=pl.DeviceIdType.MESH,
            )

        for k in (1, 2):
            started = []
            for b, (base, r, masks) in enumerate(_BFLY):
                mask = masks[k - 1]
                src_buf = x_ref if k == 1 else out_ref
                for i, h in enumerate(_H[b][k]):
                    d = desc(src_buf, recv_bufs[k], base, r,
                             my ^ mask ^ h, b, _RS_SLOT[k] + i, my ^ mask)
                    d.start()
                    started.append(d)
            for d in started:
                d.wait_recv()
            for b, (base, r, masks) in enumerate(_BFLY):
                for h in _H[b][k]:
                    rows = pl.ds(base + (my ^ h) * r, r)
                    if k == 1:
                        out_ref[rows, :] = x_ref[rows, :] + r1[rows, :]
                    else:
                        out_ref[rows, :] = out_ref[rows, :] + r2[rows, :]
            for d in started:
                d.wait_send()

        started = []
        for b, (base, r, masks) in enumerate(_BFLY):
            mask = masks[2]
            for i, h in enumerate(_H[b][2]):
                d = desc(out_ref, r3, base, r,
                         my ^ h, b, _MERGE_SLOT + i, my ^ mask)
                d.start()
                started.append(d)
        for d in started:
            d.wait_recv()
        for b, (base, r, masks) in enumerate(_BFLY):
            for h in _H[b][2]:
                rows = pl.ds(base + (my ^ h) * r, r)
                out_ref[rows, :] = out_ref[rows, :] + r3[rows, :]
        for d in started:
            d.wait_send()

        for j in (2, 3):
            started = []
            for b, (base, r, masks) in enumerate(_BFLY):
                mask = masks[3 - j]
                for i, o in enumerate(_O[b][j]):
                    d = desc(out_ref, out_ref, base, r,
                             my ^ o, b, _AG_SLOT[j] + i, my ^ mask)
                    d.start()
                    started.append(d)
            for d in started:
                d.wait_recv()
            for d in started:
                d.wait_send()

    return pl.pallas_call(
        body,
        out_shape=jax.ShapeDtypeStruct((m, n), x.dtype),
        in_specs=[pl.BlockSpec(memory_space=pltpu.VMEM)],
        out_specs=pl.BlockSpec(memory_space=pltpu.VMEM),
        scratch_shapes=[
            pltpu.VMEM((m, n), x.dtype),
            pltpu.VMEM((m, n), x.dtype),
            pltpu.VMEM((m, n), x.dtype),
            pltpu.SemaphoreType.DMA((3, 14)),
            pltpu.SemaphoreType.DMA((3, 14)),
        ],
        compiler_params=pltpu.CompilerParams(collective_id=0),
    )(x)


# device time: 39489 ns/iter; 2.8541x vs baseline; 1.1249x over previous
import jax
import jax.numpy as jnp
from jax import lax
from jax.experimental import pallas as pl
from jax.experimental.pallas import tpu as pltpu

N_DEV = 8
MASKS = (1, 3, 4)

_BFLY = (
    (0, 88, (1, 3, 4)),
    (704, 80, (3, 4, 1)),
    (1344, 88, (4, 1, 3)),
)


def kernel(x):
    m, n = x.shape

    def body(x_ref, out_ref, r1, r2, r3, send_sems, recv_sems):
        my = lax.axis_index("i")

        barrier_sem = pltpu.get_barrier_semaphore()
        for mask in MASKS:
            pl.semaphore_signal(
                barrier_sem, inc=1,
                device_id=(my ^ mask,), device_id_type=pl.DeviceIdType.MESH,
            )
        pl.semaphore_wait(barrier_sem, len(MASKS))

        def rows(b, c):
            base, r, _ = _BFLY[b]
            return pl.ds(base + c * r, r)

        descs = {}

        def start(b, src_buf, dst_buf, c, slot, target):
            d = pltpu.make_async_remote_copy(
                src_ref=src_buf.at[rows(b, c), :],
                dst_ref=dst_buf.at[rows(b, c), :],
                send_sem=send_sems.at[b, slot],
                recv_sem=recv_sems.at[b, slot],
                device_id=(target,),
                device_id_type=pl.DeviceIdType.MESH,
            )
            d.start()
            descs[(b, slot)] = d

        def masks(b):
            return _BFLY[b][2]

        B = range(len(_BFLY))

        for b in B:
            m1, m2, m3 = masks(b)
            for i, h in enumerate((m2, m2 ^ m3, 0, m3)):
                start(b, x_ref, r1, my ^ m1 ^ h, i, my ^ m1)

        for b in B:
            m1, m2, m3 = masks(b)
            for i, h in ((0, m2), (1, m2 ^ m3)):
                descs[(b, i)].wait_recv()
                c = my ^ h
                out_ref[rows(b, c), :] = x_ref[rows(b, c), :] + r1[rows(b, c), :]
        for b in B:
            m1, m2, m3 = masks(b)
            for i, h in enumerate((0, m3)):
                start(b, out_ref, r2, my ^ m2 ^ h, 4 + i, my ^ m2)

        for b in B:
            m1, m2, m3 = masks(b)
            for i, h in ((2, 0), (3, m3)):
                descs[(b, i)].wait_recv()
                c = my ^ h
                out_ref[rows(b, c), :] = x_ref[rows(b, c), :] + r1[rows(b, c), :]

        for b in B:
            m1, m2, m3 = masks(b)
            descs[(b, 4)].wait_recv()
            out_ref[rows(b, my), :] = out_ref[rows(b, my), :] + r2[rows(b, my), :]
        for b in B:
            m1, m2, m3 = masks(b)
            start(b, out_ref, r3, my, 6, my ^ m3)

        for b in B:
            m1, m2, m3 = masks(b)
            c = my ^ m3
            descs[(b, 5)].wait_recv()
            out_ref[rows(b, c), :] = out_ref[rows(b, c), :] + r2[rows(b, c), :]
        for b in B:
            m1, m2, m3 = masks(b)
            start(b, out_ref, r3, my ^ m3, 7, my ^ m3)

        for b in B:
            m1, m2, m3 = masks(b)
            c = my ^ m3
            descs[(b, 6)].wait_recv()
            descs[(b, 7)].wait_send()
            out_ref[rows(b, c), :] = out_ref[rows(b, c), :] + r3[rows(b, c), :]
        for b in B:
            m1, m2, m3 = masks(b)
            start(b, out_ref, out_ref, my ^ m3, 9, my ^ m2)
            start(b, out_ref, out_ref, my ^ m3, 11, my ^ m1)

        for b in B:
            m1, m2, m3 = masks(b)
            descs[(b, 7)].wait_recv()
            descs[(b, 6)].wait_send()
            out_ref[rows(b, my), :] = out_ref[rows(b, my), :] + r3[rows(b, my), :]
        for b in B:
            m1, m2, m3 = masks(b)
            start(b, out_ref, out_ref, my, 8, my ^ m2)
            start(b, out_ref, out_ref, my, 10, my ^ m1)

        for b in B:
            m1, m2, m3 = masks(b)
            descs[(b, 8)].wait_recv()
            start(b, out_ref, out_ref, my ^ m2, 12, my ^ m1)
        for b in B:
            m1, m2, m3 = masks(b)
            descs[(b, 9)].wait_recv()
            start(b, out_ref, out_ref, my ^ m2 ^ m3, 13, my ^ m1)

        for slot in (11, 10, 12, 13):
            for b in B:
                descs[(b, slot)].wait_recv()

        for (b, slot), d in descs.items():
            if slot not in (6, 7):
                d.wait_send()

    return pl.pallas_call(
        body,
        out_shape=jax.ShapeDtypeStruct((m, n), x.dtype),
        in_specs=[pl.BlockSpec(memory_space=pltpu.VMEM)],
        out_specs=pl.BlockSpec(memory_space=pltpu.VMEM),
        scratch_shapes=[
            pltpu.VMEM((m, n), x.dtype),
            pltpu.VMEM((m, n), x.dtype),
            pltpu.VMEM((m, n), x.dtype),
            pltpu.SemaphoreType.DMA((3, 14)),
            pltpu.SemaphoreType.DMA((3, 14)),
        ],
        compiler_params=pltpu.CompilerParams(collective_id=0),
    )(x)


# device time: 39386 ns/iter; 2.8615x vs baseline; 1.0026x over previous
import jax
import jax.numpy as jnp
from jax import lax
from jax.experimental import pallas as pl
from jax.experimental.pallas import tpu as pltpu

N_DEV = 8
MASKS = (1, 3, 4)

_BFLY = (
    (0, 88, (1, 3, 4)),
    (704, 80, (3, 4, 1)),
    (1344, 88, (4, 1, 3)),
)


def kernel(x):
    m, n = x.shape

    def body(x_ref, out_ref, r1, r2, r3, send_sems, recv_sems):
        my = lax.axis_index("i")

        barrier_sem = pltpu.get_barrier_semaphore()
        for mask in MASKS:
            pl.semaphore_signal(
                barrier_sem, inc=1,
                device_id=(my ^ mask,), device_id_type=pl.DeviceIdType.MESH,
            )
        pl.semaphore_wait(barrier_sem, len(MASKS))

        def rows(b, c):
            base, r, _ = _BFLY[b]
            return pl.ds(base + c * r, r)

        descs = {}

        def start(b, src_buf, dst_buf, c, slot, target):
            d = pltpu.make_async_remote_copy(
                src_ref=src_buf.at[rows(b, c), :],
                dst_ref=dst_buf.at[rows(b, c), :],
                send_sem=send_sems.at[b, slot],
                recv_sem=recv_sems.at[b, slot],
                device_id=(target,),
                device_id_type=pl.DeviceIdType.MESH,
            )
            d.start()
            descs[(b, slot)] = d

        def masks(b):
            return _BFLY[b][2]

        B = range(len(_BFLY))

        for b in B:
            m1, m2, m3 = masks(b)
            for i, h in enumerate((m2, m2 ^ m3, 0, m3)):
                start(b, x_ref, r1, my ^ m1 ^ h, i, my ^ m1)

        for b in B:
            m1, m2, m3 = masks(b)
            for i, h in ((0, m2), (1, m2 ^ m3)):
                descs[(b, i)].wait_recv()
                c = my ^ h
                out_ref[rows(b, c), :] = x_ref[rows(b, c), :] + r1[rows(b, c), :]
            for i, h in enumerate((0, m3)):
                start(b, out_ref, r2, my ^ m2 ^ h, 4 + i, my ^ m2)

        for b in B:
            m1, m2, m3 = masks(b)
            for i, h in ((2, 0), (3, m3)):
                descs[(b, i)].wait_recv()
                c = my ^ h
                out_ref[rows(b, c), :] = x_ref[rows(b, c), :] + r1[rows(b, c), :]

        for b in B:
            m1, m2, m3 = masks(b)
            descs[(b, 4)].wait_recv()
            out_ref[rows(b, my), :] = out_ref[rows(b, my), :] + r2[rows(b, my), :]
            start(b, out_ref, r3, my, 6, my ^ m3)

        for b in B:
            m1, m2, m3 = masks(b)
            c = my ^ m3
            descs[(b, 5)].wait_recv()
            out_ref[rows(b, c), :] = out_ref[rows(b, c), :] + r2[rows(b, c), :]
            start(b, out_ref, r3, my ^ m3, 7, my ^ m3)

        for b in B:
            m1, m2, m3 = masks(b)
            c = my ^ m3
            descs[(b, 6)].wait_recv()
            descs[(b, 7)].wait_send()
            out_ref[rows(b, c), :] = out_ref[rows(b, c), :] + r3[rows(b, c), :]
            start(b, out_ref, out_ref, my ^ m3, 9, my ^ m2)
            start(b, out_ref, out_ref, my ^ m3, 11, my ^ m1)

        for b in B:
            m1, m2, m3 = masks(b)
            descs[(b, 7)].wait_recv()
            descs[(b, 6)].wait_send()
            out_ref[rows(b, my), :] = out_ref[rows(b, my), :] + r3[rows(b, my), :]
            start(b, out_ref, out_ref, my, 8, my ^ m2)
            start(b, out_ref, out_ref, my, 10, my ^ m1)

        for b in B:
            m1, m2, m3 = masks(b)
            descs[(b, 8)].wait_recv()
            start(b, out_ref, out_ref, my ^ m2, 12, my ^ m1)
        for b in B:
            m1, m2, m3 = masks(b)
            descs[(b, 9)].wait_recv()
            start(b, out_ref, out_ref, my ^ m2 ^ m3, 13, my ^ m1)

        for slot in (11, 10, 12, 13):
            for b in B:
                descs[(b, slot)].wait_recv()

        for (b, slot), d in descs.items():
            if slot not in (6, 7):
                d.wait_send()

    return pl.pallas_call(
        body,
        out_shape=jax.ShapeDtypeStruct((m, n), x.dtype),
        in_specs=[pl.BlockSpec(memory_space=pltpu.VMEM)],
        out_specs=pl.BlockSpec(memory_space=pltpu.VMEM),
        scratch_shapes=[
            pltpu.VMEM((m, n), x.dtype),
            pltpu.VMEM((m, n), x.dtype),
            pltpu.VMEM((m, n), x.dtype),
            pltpu.SemaphoreType.DMA((3, 14)),
            pltpu.SemaphoreType.DMA((3, 14)),
        ],
        compiler_params=pltpu.CompilerParams(collective_id=0),
    )(x)
